# baseline (speedup 1.0000x reference)
import functools

import jax
import jax.numpy as jnp
from jax import lax
from jax.experimental import pallas as pl
from jax.experimental.pallas import tpu as pltpu
from jax.experimental.pallas import tpu_sc as plsc

_B, _H, _W, _D = 32, 16, 16, 256
_DEPTH = 4
_E = 8192
_N = _B * _H * _W

_TM = 512
_TN = 1024
_TF = 1024


def _dist_argmin(r, cbt_ref, j):
    cbt = cbt_ref[...]
    scores = lax.dot_general(r, cbt, (((1,), (0,)), ((), ())),
                             preferred_element_type=jnp.float32)
    csq = jnp.sum(cbt * cbt, axis=0, keepdims=True)
    rsq = jnp.sum(r * r, axis=1, keepdims=True)
    dist = (rsq + csq) - 2.0 * scores
    tmin = jnp.min(dist, axis=1, keepdims=True)
    col = lax.broadcasted_iota(jnp.int32, dist.shape, 1)
    targ = jnp.min(jnp.where(dist == tmin, col, jnp.int32(2**30)),
                   axis=1, keepdims=True)
    return tmin, targ + j * _TN


def _argmin_update(minv, mini, tmin, targ, j):
    @pl.when(j == 0)
    def _():
        minv[...] = tmin
        mini[...] = targ

    @pl.when(j > 0)
    def _():
        better = tmin < minv[...]
        mini[...] = jnp.where(better, targ, mini[...])
        minv[...] = jnp.where(better, tmin, minv[...])


def _argmin0_body(r_ref, cbt_ref, idx_ref, minv, mini):
    j = pl.program_id(1)
    tmin, targ = _dist_argmin(r_ref[...], cbt_ref, j)
    _argmin_update(minv, mini, tmin, targ, j)

    @pl.when(j == pl.num_programs(1) - 1)
    def _():
        idx_ref[...] = mini[...]


def _argmin_sub_body(r_ref, q_ref, cbt_ref, idx_ref, rnew_ref, minv, mini):
    j = pl.program_id(1)
    r = r_ref[...] - q_ref[...]

    @pl.when(j == 0)
    def _():
        rnew_ref[...] = r

    tmin, targ = _dist_argmin(r, cbt_ref, j)
    _argmin_update(minv, mini, tmin, targ, j)

    @pl.when(j == pl.num_programs(1) - 1)
    def _():
        idx_ref[...] = mini[...]


_TT = 512


def _transpose_body(in_ref, out_ref):
    out_ref[...] = in_ref[...].T


def _transpose(cb):
    return pl.pallas_call(
        _transpose_body,
        grid=(_E // _TT,),
        in_specs=[pl.BlockSpec((_TT, _D), lambda i: (i, 0))],
        out_specs=pl.BlockSpec((_D, _TT), lambda i: (0, i)),
        out_shape=jax.ShapeDtypeStruct((_D, _E), jnp.float32),
    )(cb)


def _argmin0(r, cbt):
    return pl.pallas_call(
        _argmin0_body,
        grid=(_N // _TM, _E // _TN),
        in_specs=[pl.BlockSpec((_TM, _D), lambda i, j: (i, 0)),
                  pl.BlockSpec((_D, _TN), lambda i, j: (0, j))],
        out_specs=pl.BlockSpec((_TM, 1), lambda i, j: (i, 0)),
        out_shape=jax.ShapeDtypeStruct((_N, 1), jnp.int32),
        scratch_shapes=[pltpu.VMEM((_TM, 1), jnp.float32),
                        pltpu.VMEM((_TM, 1), jnp.int32)],
        compiler_params=pltpu.CompilerParams(
            dimension_semantics=("arbitrary", "arbitrary")),
    )(r, cbt)


def _argmin_sub(r_prev, q_prev, cbt):
    return pl.pallas_call(
        _argmin_sub_body,
        grid=(_N // _TM, _E // _TN),
        in_specs=[pl.BlockSpec((_TM, _D), lambda i, j: (i, 0)),
                  pl.BlockSpec((_TM, _D), lambda i, j: (i, 0)),
                  pl.BlockSpec((_D, _TN), lambda i, j: (0, j))],
        out_specs=[pl.BlockSpec((_TM, 1), lambda i, j: (i, 0)),
                   pl.BlockSpec((_TM, _D), lambda i, j: (i, 0))],
        out_shape=[jax.ShapeDtypeStruct((_N, 1), jnp.int32),
                   jax.ShapeDtypeStruct((_N, _D), jnp.float32)],
        scratch_shapes=[pltpu.VMEM((_TM, 1), jnp.float32),
                        pltpu.VMEM((_TM, 1), jnp.int32)],
        compiler_params=pltpu.CompilerParams(
            dimension_semantics=("arbitrary", "arbitrary")),
    )(r_prev, q_prev, cbt)


def _sc_gather(cb, idx):
    info = plsc.get_sparse_core_info()
    nc, ns = info.num_cores, info.num_subcores
    nw = nc * ns
    bpw = _N // nw
    chunk = 128
    nchunk = bpw // chunk
    mesh = plsc.VectorSubcoreMesh(core_axis_name="c", subcore_axis_name="s")

    @functools.partial(
        pl.kernel, mesh=mesh,
        out_type=jax.ShapeDtypeStruct((_N, _D), jnp.float32),
        scratch_types=[pltpu.VMEM((nchunk, chunk), jnp.int32),
                       pltpu.VMEM((bpw, _D), jnp.float32),
                       pltpu.SemaphoreType.DMA],
    )
    def k(cb_hbm, idx_hbm, out_hbm, idx_v, rows_v, sem):
        wid = lax.axis_index("s") * nc + lax.axis_index("c")
        base = wid * bpw
        for c in range(nchunk):
            pltpu.sync_copy(idx_hbm.at[pl.ds(base + c * chunk, chunk)],
                            idx_v.at[c])
        copies = [
            pltpu.async_copy(cb_hbm.at[idx_v.at[c]],
                             rows_v.at[pl.ds(c * chunk, chunk)], sem)
            for c in range(nchunk)
        ]
        for cp in copies:
            cp.wait()
        pltpu.sync_copy(rows_v, out_hbm.at[pl.ds(base, bpw)])

    return k(cb, idx)


def _finalize(x, r1, r2, r3, q3):
    def body(x_ref, r1_ref, r2_ref, r3_ref, q3_ref, quants_ref, loss_ref, acc):
        i = pl.program_id(0)

        @pl.when(i == 0)
        def _():
            acc[0] = 0.0
            acc[1] = 0.0
            acc[2] = 0.0
            acc[3] = 0.0

        r4 = r3_ref[...] - q3_ref[...]
        quants_ref[...] = x_ref[...] - r4
        acc[0] += jnp.sum(r1_ref[...] * r1_ref[...])
        acc[1] += jnp.sum(r2_ref[...] * r2_ref[...])
        acc[2] += jnp.sum(r3_ref[...] * r3_ref[...])
        acc[3] += jnp.sum(r4 * r4)

        @pl.when(i == pl.num_programs(0) - 1)
        def _():
            loss_ref[0, 0] = (acc[0] + acc[1] + acc[2] + acc[3]) / (4.0 * _N * _D)

    return pl.pallas_call(
        body,
        grid=(_N // _TF,),
        in_specs=[pl.BlockSpec((_TF, _D), lambda i: (i, 0))] * 5,
        out_specs=[pl.BlockSpec((_TF, _D), lambda i: (i, 0)),
                   pl.BlockSpec((1, 1), lambda i: (0, 0),
                                memory_space=pltpu.SMEM)],
        out_shape=[jax.ShapeDtypeStruct((_N, _D), jnp.float32),
                   jax.ShapeDtypeStruct((1, 1), jnp.float32)],
        scratch_shapes=[pltpu.SMEM((4,), jnp.float32)],
        compiler_params=pltpu.CompilerParams(
            dimension_semantics=("arbitrary",)),
    )(x, r1, r2, r3, q3)


def kernel(x, codebooks):
    x_flat = x.reshape(_N, _D)
    cbs = [codebooks[lvl] for lvl in range(_DEPTH)]
    cbts = [_transpose(cb) for cb in cbs]

    idx0 = _argmin0(x_flat, cbts[0])
    q0 = _sc_gather(cbs[0], idx0.reshape(_N))
    idx1, r1 = _argmin_sub(x_flat, q0, cbts[1])
    q1 = _sc_gather(cbs[1], idx1.reshape(_N))
    idx2, r2 = _argmin_sub(r1, q1, cbts[2])
    q2 = _sc_gather(cbs[2], idx2.reshape(_N))
    idx3, r3 = _argmin_sub(r2, q2, cbts[3])
    q3 = _sc_gather(cbs[3], idx3.reshape(_N))

    quants_flat, loss = _finalize(x_flat, r1, r2, r3, q3)
    quants = quants_flat.reshape(_B, _H, _W, _D)
    codes = jnp.concatenate([idx0, idx1, idx2, idx3],
                            axis=1).reshape(_B, _H, _W, _DEPTH)
    return quants, loss[0, 0], codes


# speedup vs baseline: 1.2500x; 1.2500x over previous
import functools

import jax
import jax.numpy as jnp
from jax import lax
from jax.experimental import pallas as pl
from jax.experimental.pallas import tpu as pltpu
from jax.experimental.pallas import tpu_sc as plsc

_B, _H, _W, _D = 32, 16, 16, 256
_DEPTH = 4
_E = 8192
_N = _B * _H * _W

_TM = 512
_TN = 2048
_TF = 1024


def _dist_argmin(r, cbt_ref, j):
    cbt = cbt_ref[...]
    scores = lax.dot_general(r, cbt, (((1,), (0,)), ((), ())),
                             preferred_element_type=jnp.float32)
    csq = jnp.sum(cbt * cbt, axis=0, keepdims=True)
    rsq = jnp.sum(r * r, axis=1, keepdims=True)
    dist = (rsq + csq) - 2.0 * scores
    tmin = jnp.min(dist, axis=1, keepdims=True)
    col = lax.broadcasted_iota(jnp.int32, dist.shape, 1)
    targ = jnp.min(jnp.where(dist == tmin, col, jnp.int32(2**30)),
                   axis=1, keepdims=True)
    return tmin, targ + j * _TN


def _argmin_update(minv, mini, tmin, targ, j):
    @pl.when(j == 0)
    def _():
        minv[...] = tmin
        mini[...] = targ

    @pl.when(j > 0)
    def _():
        better = tmin < minv[...]
        mini[...] = jnp.where(better, targ, mini[...])
        minv[...] = jnp.where(better, tmin, minv[...])


def _argmin0_body(r_ref, cbt_ref, idx_ref, minv, mini):
    j = pl.program_id(1)
    tmin, targ = _dist_argmin(r_ref[...], cbt_ref, j)
    _argmin_update(minv, mini, tmin, targ, j)

    @pl.when(j == pl.num_programs(1) - 1)
    def _():
        idx_ref[...] = mini[...]


def _argmin_sub_body(r_ref, q_ref, cbt_ref, idx_ref, rnew_ref, minv, mini):
    j = pl.program_id(1)
    r = r_ref[...] - q_ref[...]

    @pl.when(j == 0)
    def _():
        rnew_ref[...] = r

    tmin, targ = _dist_argmin(r, cbt_ref, j)
    _argmin_update(minv, mini, tmin, targ, j)

    @pl.when(j == pl.num_programs(1) - 1)
    def _():
        idx_ref[...] = mini[...]


_TT = 512


def _transpose_body(in_ref, out_ref):
    out_ref[...] = in_ref[...].T


def _transpose(cb):
    return pl.pallas_call(
        _transpose_body,
        grid=(_E // _TT,),
        in_specs=[pl.BlockSpec((_TT, _D), lambda i: (i, 0))],
        out_specs=pl.BlockSpec((_D, _TT), lambda i: (0, i)),
        out_shape=jax.ShapeDtypeStruct((_D, _E), jnp.float32),
    )(cb)


def _argmin0(r, cbt):
    return pl.pallas_call(
        _argmin0_body,
        grid=(_N // _TM, _E // _TN),
        in_specs=[pl.BlockSpec((_TM, _D), lambda i, j: (i, 0)),
                  pl.BlockSpec((_D, _TN), lambda i, j: (0, j))],
        out_specs=pl.BlockSpec((_TM, 1), lambda i, j: (i, 0)),
        out_shape=jax.ShapeDtypeStruct((_N, 1), jnp.int32),
        scratch_shapes=[pltpu.VMEM((_TM, 1), jnp.float32),
                        pltpu.VMEM((_TM, 1), jnp.int32)],
        compiler_params=pltpu.CompilerParams(
            dimension_semantics=("arbitrary", "arbitrary")),
    )(r, cbt)


def _argmin_sub(r_prev, q_prev, cbt):
    return pl.pallas_call(
        _argmin_sub_body,
        grid=(_N // _TM, _E // _TN),
        in_specs=[pl.BlockSpec((_TM, _D), lambda i, j: (i, 0)),
                  pl.BlockSpec((_TM, _D), lambda i, j: (i, 0)),
                  pl.BlockSpec((_D, _TN), lambda i, j: (0, j))],
        out_specs=[pl.BlockSpec((_TM, 1), lambda i, j: (i, 0)),
                   pl.BlockSpec((_TM, _D), lambda i, j: (i, 0))],
        out_shape=[jax.ShapeDtypeStruct((_N, 1), jnp.int32),
                   jax.ShapeDtypeStruct((_N, _D), jnp.float32)],
        scratch_shapes=[pltpu.VMEM((_TM, 1), jnp.float32),
                        pltpu.VMEM((_TM, 1), jnp.int32)],
        compiler_params=pltpu.CompilerParams(
            dimension_semantics=("arbitrary", "arbitrary")),
    )(r_prev, q_prev, cbt)


def _sc_gather(cb, idx):
    info = plsc.get_sparse_core_info()
    nc, ns = info.num_cores, info.num_subcores
    nw = nc * ns
    bpw = _N // nw
    chunk = 128
    nchunk = bpw // chunk
    mesh = plsc.VectorSubcoreMesh(core_axis_name="c", subcore_axis_name="s")

    @functools.partial(
        pl.kernel, mesh=mesh,
        out_type=jax.ShapeDtypeStruct((_N, _D), jnp.float32),
        scratch_types=[pltpu.VMEM((nchunk, chunk), jnp.int32),
                       pltpu.VMEM((bpw, _D), jnp.float32),
                       pltpu.SemaphoreType.DMA],
    )
    def k(cb_hbm, idx_hbm, out_hbm, idx_v, rows_v, sem):
        wid = lax.axis_index("s") * nc + lax.axis_index("c")
        base = wid * bpw
        for c in range(nchunk):
            pltpu.sync_copy(idx_hbm.at[pl.ds(base + c * chunk, chunk)],
                            idx_v.at[c])
        copies = [
            pltpu.async_copy(cb_hbm.at[idx_v.at[c]],
                             rows_v.at[pl.ds(c * chunk, chunk)], sem)
            for c in range(nchunk)
        ]
        for cp in copies:
            cp.wait()
        pltpu.sync_copy(rows_v, out_hbm.at[pl.ds(base, bpw)])

    return k(cb, idx)


def _finalize(x, r1, r2, r3, q3):
    def body(x_ref, r1_ref, r2_ref, r3_ref, q3_ref, quants_ref, loss_ref, acc):
        i = pl.program_id(0)

        @pl.when(i == 0)
        def _():
            acc[0] = 0.0
            acc[1] = 0.0
            acc[2] = 0.0
            acc[3] = 0.0

        r4 = r3_ref[...] - q3_ref[...]
        quants_ref[...] = x_ref[...] - r4
        acc[0] += jnp.sum(r1_ref[...] * r1_ref[...])
        acc[1] += jnp.sum(r2_ref[...] * r2_ref[...])
        acc[2] += jnp.sum(r3_ref[...] * r3_ref[...])
        acc[3] += jnp.sum(r4 * r4)

        @pl.when(i == pl.num_programs(0) - 1)
        def _():
            loss_ref[0, 0] = (acc[0] + acc[1] + acc[2] + acc[3]) / (4.0 * _N * _D)

    return pl.pallas_call(
        body,
        grid=(_N // _TF,),
        in_specs=[pl.BlockSpec((_TF, _D), lambda i: (i, 0))] * 5,
        out_specs=[pl.BlockSpec((_TF, _D), lambda i: (i, 0)),
                   pl.BlockSpec((1, 1), lambda i: (0, 0),
                                memory_space=pltpu.SMEM)],
        out_shape=[jax.ShapeDtypeStruct((_N, _D), jnp.float32),
                   jax.ShapeDtypeStruct((1, 1), jnp.float32)],
        scratch_shapes=[pltpu.SMEM((4,), jnp.float32)],
        compiler_params=pltpu.CompilerParams(
            dimension_semantics=("arbitrary",)),
    )(x, r1, r2, r3, q3)


def kernel(x, codebooks):
    x_flat = x.reshape(_N, _D)
    cbs = [codebooks[lvl] for lvl in range(_DEPTH)]
    cbts = [_transpose(cb) for cb in cbs]

    idx0 = _argmin0(x_flat, cbts[0])
    q0 = _sc_gather(cbs[0], idx0.reshape(_N))
    idx1, r1 = _argmin_sub(x_flat, q0, cbts[1])
    q1 = _sc_gather(cbs[1], idx1.reshape(_N))
    idx2, r2 = _argmin_sub(r1, q1, cbts[2])
    q2 = _sc_gather(cbs[2], idx2.reshape(_N))
    idx3, r3 = _argmin_sub(r2, q2, cbts[3])
    q3 = _sc_gather(cbs[3], idx3.reshape(_N))

    quants_flat, loss = _finalize(x_flat, r1, r2, r3, q3)
    quants = quants_flat.reshape(_B, _H, _W, _D)
    codes = jnp.concatenate([idx0, idx1, idx2, idx3],
                            axis=1).reshape(_B, _H, _W, _DEPTH)
    return quants, loss[0, 0], codes


# speedup vs baseline: 1.3558x; 1.0846x over previous
import functools

import jax
import jax.numpy as jnp
from jax import lax
from jax.experimental import pallas as pl
from jax.experimental.pallas import tpu as pltpu
from jax.experimental.pallas import tpu_sc as plsc

_B, _H, _W, _D = 32, 16, 16, 256
_DEPTH = 4
_E = 8192
_N = _B * _H * _W

_TM = 512
_TN = 4096
_TF = 1024


def _dist_argmin(r, cbt_ref, j):
    cbt = cbt_ref[...]
    scores = lax.dot_general(r, cbt, (((1,), (0,)), ((), ())),
                             preferred_element_type=jnp.float32)
    csq = jnp.sum(cbt * cbt, axis=0, keepdims=True)
    rsq = jnp.sum(r * r, axis=1, keepdims=True)
    dist = (rsq + csq) - 2.0 * scores
    tmin = jnp.min(dist, axis=1, keepdims=True)
    col = lax.broadcasted_iota(jnp.int32, dist.shape, 1)
    targ = jnp.min(jnp.where(dist == tmin, col, jnp.int32(2**30)),
                   axis=1, keepdims=True)
    return tmin, targ + j * _TN


def _argmin_update(minv, mini, tmin, targ, j):
    @pl.when(j == 0)
    def _():
        minv[...] = tmin
        mini[...] = targ

    @pl.when(j > 0)
    def _():
        better = tmin < minv[...]
        mini[...] = jnp.where(better, targ, mini[...])
        minv[...] = jnp.where(better, tmin, minv[...])


def _argmin0_body(r_ref, cbt_ref, idx_ref, minv, mini):
    j = pl.program_id(1)
    tmin, targ = _dist_argmin(r_ref[...], cbt_ref, j)
    _argmin_update(minv, mini, tmin, targ, j)

    @pl.when(j == pl.num_programs(1) - 1)
    def _():
        idx_ref[...] = mini[...]


def _argmin_sub_body(r_ref, q_ref, cbt_ref, idx_ref, rnew_ref, minv, mini):
    j = pl.program_id(1)
    r = r_ref[...] - q_ref[...]

    @pl.when(j == 0)
    def _():
        rnew_ref[...] = r

    tmin, targ = _dist_argmin(r, cbt_ref, j)
    _argmin_update(minv, mini, tmin, targ, j)

    @pl.when(j == pl.num_programs(1) - 1)
    def _():
        idx_ref[...] = mini[...]


_TT = 512


def _transpose_body(in_ref, out_ref):
    out_ref[...] = in_ref[...].T


def _transpose(cb):
    return pl.pallas_call(
        _transpose_body,
        grid=(_E // _TT,),
        in_specs=[pl.BlockSpec((_TT, _D), lambda i: (i, 0))],
        out_specs=pl.BlockSpec((_D, _TT), lambda i: (0, i)),
        out_shape=jax.ShapeDtypeStruct((_D, _E), jnp.float32),
    )(cb)


def _argmin0(r, cbt):
    return pl.pallas_call(
        _argmin0_body,
        grid=(_N // _TM, _E // _TN),
        in_specs=[pl.BlockSpec((_TM, _D), lambda i, j: (i, 0)),
                  pl.BlockSpec((_D, _TN), lambda i, j: (0, j))],
        out_specs=pl.BlockSpec((_TM, 1), lambda i, j: (i, 0)),
        out_shape=jax.ShapeDtypeStruct((_N, 1), jnp.int32),
        scratch_shapes=[pltpu.VMEM((_TM, 1), jnp.float32),
                        pltpu.VMEM((_TM, 1), jnp.int32)],
        compiler_params=pltpu.CompilerParams(
            dimension_semantics=("arbitrary", "arbitrary")),
    )(r, cbt)


def _argmin_sub(r_prev, q_prev, cbt):
    return pl.pallas_call(
        _argmin_sub_body,
        grid=(_N // _TM, _E // _TN),
        in_specs=[pl.BlockSpec((_TM, _D), lambda i, j: (i, 0)),
                  pl.BlockSpec((_TM, _D), lambda i, j: (i, 0)),
                  pl.BlockSpec((_D, _TN), lambda i, j: (0, j))],
        out_specs=[pl.BlockSpec((_TM, 1), lambda i, j: (i, 0)),
                   pl.BlockSpec((_TM, _D), lambda i, j: (i, 0))],
        out_shape=[jax.ShapeDtypeStruct((_N, 1), jnp.int32),
                   jax.ShapeDtypeStruct((_N, _D), jnp.float32)],
        scratch_shapes=[pltpu.VMEM((_TM, 1), jnp.float32),
                        pltpu.VMEM((_TM, 1), jnp.int32)],
        compiler_params=pltpu.CompilerParams(
            dimension_semantics=("arbitrary", "arbitrary")),
    )(r_prev, q_prev, cbt)


def _sc_gather(cb, idx):
    info = plsc.get_sparse_core_info()
    nc, ns = info.num_cores, info.num_subcores
    nw = nc * ns
    bpw = _N // nw
    chunk = 128
    nchunk = bpw // chunk
    mesh = plsc.VectorSubcoreMesh(core_axis_name="c", subcore_axis_name="s")

    @functools.partial(
        pl.kernel, mesh=mesh,
        out_type=jax.ShapeDtypeStruct((_N, _D), jnp.float32),
        scratch_types=[pltpu.VMEM((nchunk, chunk), jnp.int32),
                       pltpu.VMEM((bpw, _D), jnp.float32),
                       pltpu.SemaphoreType.DMA],
    )
    def k(cb_hbm, idx_hbm, out_hbm, idx_v, rows_v, sem):
        wid = lax.axis_index("s") * nc + lax.axis_index("c")
        base = wid * bpw
        for c in range(nchunk):
            pltpu.sync_copy(idx_hbm.at[pl.ds(base + c * chunk, chunk)],
                            idx_v.at[c])
        copies = [
            pltpu.async_copy(cb_hbm.at[idx_v.at[c]],
                             rows_v.at[pl.ds(c * chunk, chunk)], sem)
            for c in range(nchunk)
        ]
        for cp in copies:
            cp.wait()
        pltpu.sync_copy(rows_v, out_hbm.at[pl.ds(base, bpw)])

    return k(cb, idx)


def _finalize(x, r1, r2, r3, q3):
    def body(x_ref, r1_ref, r2_ref, r3_ref, q3_ref, quants_ref, loss_ref, acc):
        i = pl.program_id(0)

        @pl.when(i == 0)
        def _():
            acc[0] = 0.0
            acc[1] = 0.0
            acc[2] = 0.0
            acc[3] = 0.0

        r4 = r3_ref[...] - q3_ref[...]
        quants_ref[...] = x_ref[...] - r4
        acc[0] += jnp.sum(r1_ref[...] * r1_ref[...])
        acc[1] += jnp.sum(r2_ref[...] * r2_ref[...])
        acc[2] += jnp.sum(r3_ref[...] * r3_ref[...])
        acc[3] += jnp.sum(r4 * r4)

        @pl.when(i == pl.num_programs(0) - 1)
        def _():
            loss_ref[0, 0] = (acc[0] + acc[1] + acc[2] + acc[3]) / (4.0 * _N * _D)

    return pl.pallas_call(
        body,
        grid=(_N // _TF,),
        in_specs=[pl.BlockSpec((_TF, _D), lambda i: (i, 0))] * 5,
        out_specs=[pl.BlockSpec((_TF, _D), lambda i: (i, 0)),
                   pl.BlockSpec((1, 1), lambda i: (0, 0),
                                memory_space=pltpu.SMEM)],
        out_shape=[jax.ShapeDtypeStruct((_N, _D), jnp.float32),
                   jax.ShapeDtypeStruct((1, 1), jnp.float32)],
        scratch_shapes=[pltpu.SMEM((4,), jnp.float32)],
        compiler_params=pltpu.CompilerParams(
            dimension_semantics=("arbitrary",)),
    )(x, r1, r2, r3, q3)


def kernel(x, codebooks):
    x_flat = x.reshape(_N, _D)
    cbs = [codebooks[lvl] for lvl in range(_DEPTH)]
    cbts = [_transpose(cb) for cb in cbs]

    idx0 = _argmin0(x_flat, cbts[0])
    q0 = _sc_gather(cbs[0], idx0.reshape(_N))
    idx1, r1 = _argmin_sub(x_flat, q0, cbts[1])
    q1 = _sc_gather(cbs[1], idx1.reshape(_N))
    idx2, r2 = _argmin_sub(r1, q1, cbts[2])
    q2 = _sc_gather(cbs[2], idx2.reshape(_N))
    idx3, r3 = _argmin_sub(r2, q2, cbts[3])
    q3 = _sc_gather(cbs[3], idx3.reshape(_N))

    quants_flat, loss = _finalize(x_flat, r1, r2, r3, q3)
    quants = quants_flat.reshape(_B, _H, _W, _D)
    codes = jnp.concatenate([idx0, idx1, idx2, idx3],
                            axis=1).reshape(_B, _H, _W, _DEPTH)
    return quants, loss[0, 0], codes
